# baseline (speedup 1.0000x reference)
import jax
import jax.numpy as jnp
from jax.experimental import pallas as pl
from jax.experimental.pallas import tpu as pltpu

_NCHUNK = 8


def _copy_body(x_ref, o_ref, sems):
    for i in range(_NCHUNK):
        pltpu.make_async_copy(x_ref.at[i], o_ref.at[i], sems.at[i]).start()
    for i in range(_NCHUNK):
        pltpu.make_async_copy(x_ref.at[i], o_ref.at[i], sems.at[i]).wait()


def kernel(x):
    b, s, d = x.shape
    rows = b * s
    xr = x.reshape(_NCHUNK, rows // _NCHUNK, d)
    out = pl.pallas_call(
        _copy_body,
        out_shape=jax.ShapeDtypeStruct(xr.shape, xr.dtype),
        in_specs=[pl.BlockSpec(memory_space=pltpu.HBM)],
        out_specs=pl.BlockSpec(memory_space=pltpu.HBM),
        scratch_shapes=[pltpu.SemaphoreType.DMA((_NCHUNK,))],
    )(xr)
    return out.reshape(b, s, d)


# speedup vs baseline: 49.0197x; 49.0197x over previous
import jax
import jax.numpy as jnp
from jax.experimental import pallas as pl
from jax.experimental.pallas import tpu as pltpu

_BLOCK_ROWS = 1024


def _copy_body(x_ref, o_ref):
    o_ref[...] = x_ref[...]


def kernel(x):
    b, s, d = x.shape
    rows = b * s
    xr = x.reshape(rows, d)
    grid = rows // _BLOCK_ROWS
    out = pl.pallas_call(
        _copy_body,
        out_shape=jax.ShapeDtypeStruct(xr.shape, xr.dtype),
        grid=(grid,),
        in_specs=[pl.BlockSpec((_BLOCK_ROWS, d), lambda i: (i, 0))],
        out_specs=pl.BlockSpec((_BLOCK_ROWS, d), lambda i: (i, 0)),
        compiler_params=pltpu.CompilerParams(
            dimension_semantics=("arbitrary",),
        ),
    )(xr)
    return out.reshape(b, s, d)
